# baseline (speedup 1.0000x reference)
import functools

import jax
import jax.numpy as jnp
from jax import lax
from jax.experimental import pallas as pl
from jax.experimental.pallas import tpu as pltpu
from jax.experimental.pallas import tpu_sc as plsc


def kernel(x, emb_table):
    B, S, D = x.shape
    info = plsc.get_sparse_core_info()
    NC, NS = info.num_cores, info.num_subcores
    NW = NC * NS
    pos_per_w = S // NW
    CH = 16
    nch = pos_per_w // CH
    CE = CH * D
    nsteps = nch * B

    xf = x.reshape(B * S * D)
    ef = emb_table.reshape(S * D)
    mesh = plsc.VectorSubcoreMesh(core_axis_name="c", subcore_axis_name="s")

    @functools.partial(
        pl.kernel,
        mesh=mesh,
        out_type=jax.ShapeDtypeStruct((B * S * D,), jnp.float32),
        scratch_types=[
            pltpu.VMEM((2, CE), jnp.float32),
            pltpu.VMEM((2, CE), jnp.float32),
            pltpu.SemaphoreType.DMA,
            pltpu.SemaphoreType.DMA,
            pltpu.SemaphoreType.DMA,
            pltpu.SemaphoreType.DMA,
            pltpu.SemaphoreType.DMA,
            pltpu.SemaphoreType.DMA,
        ],
    )
    def sc_add(x_hbm, emb_hbm, out_hbm, eb, xb, se0, se1, sx0, sx1, so0, so1):
        cid = lax.axis_index("c")
        sid = lax.axis_index("s")
        wid = sid * NC + cid
        base = wid * pos_per_w
        sxs, sos, ses = (sx0, sx1), (so0, so1), (se0, se1)

        def embload(ci):
            off = (base + ci * CH) * D
            return pltpu.async_copy(
                emb_hbm.at[pl.ds(off, CE)], eb.at[ci % 2], ses[ci % 2])

        def xload(k):
            ci, b = divmod(k, B)
            off = (b * S + base + ci * CH) * D
            return pltpu.async_copy(
                x_hbm.at[pl.ds(off, CE)], xb.at[k % 2], sxs[k % 2])

        hx, he, ho = {}, {}, {}
        he[0] = embload(0)
        hx[0] = xload(0)
        for k in range(nsteps):
            ci, b = divmod(k, B)
            if k + 1 < nsteps:
                if k >= 1:
                    ho[k - 1].wait()
                hx[k + 1] = xload(k + 1)
            if b == 0 and ci + 1 < nch:
                he[ci + 1] = embload(ci + 1)
            hx[k].wait()
            if b == 0:
                he[ci].wait()
            xbk = xb.at[k % 2]
            ebk = eb.at[ci % 2]

            def addbody(j, carry):
                off = j * 128
                for u in range(8):
                    o = off + u * 16
                    xbk[pl.ds(o, 16)] = xbk[pl.ds(o, 16)] + ebk[pl.ds(o, 16)]
                return carry

            lax.fori_loop(0, CE // 128, addbody, 0)
            off = (b * S + base + ci * CH) * D
            ho[k] = pltpu.async_copy(
                xbk, out_hbm.at[pl.ds(off, CE)], sos[k % 2])
        ho[nsteps - 2].wait()
        ho[nsteps - 1].wait()

    return sc_add(xf, ef).reshape(B, S, D)


# speedup vs baseline: 2.2467x; 2.2467x over previous
import functools

import jax
import jax.numpy as jnp
from jax import lax
from jax.experimental import pallas as pl
from jax.experimental.pallas import tpu as pltpu
from jax.experimental.pallas import tpu_sc as plsc


def kernel(x, emb_table):
    B, S, D = x.shape
    info = plsc.get_sparse_core_info()
    NC, NS = info.num_cores, info.num_subcores
    NW = NC * NS
    pos_per_w = S // NW
    CH = 16
    nch = pos_per_w // CH
    nsteps = nch * B

    x2 = x.reshape(B * S, D)
    mesh = plsc.VectorSubcoreMesh(core_axis_name="c", subcore_axis_name="s")

    @functools.partial(
        pl.kernel,
        mesh=mesh,
        out_type=jax.ShapeDtypeStruct((B * S, D), jnp.float32),
        compiler_params=pltpu.CompilerParams(use_tc_tiling_on_sc=True),
        scratch_types=[
            pltpu.VMEM((2, CH, D), jnp.float32),
            pltpu.VMEM((2, CH, D), jnp.float32),
            pltpu.SemaphoreType.DMA,
            pltpu.SemaphoreType.DMA,
            pltpu.SemaphoreType.DMA,
            pltpu.SemaphoreType.DMA,
            pltpu.SemaphoreType.DMA,
            pltpu.SemaphoreType.DMA,
        ],
    )
    def sc_add(x_hbm, emb_hbm, out_hbm, eb, xb, se0, se1, sx0, sx1, so0, so1):
        cid = lax.axis_index("c")
        sid = lax.axis_index("s")
        wid = sid * NC + cid
        base = wid * pos_per_w
        sxs, sos, ses = (sx0, sx1), (so0, so1), (se0, se1)

        def embload(ci):
            return pltpu.async_copy(
                emb_hbm.at[pl.ds(base + ci * CH, CH)], eb.at[ci % 2],
                ses[ci % 2])

        def xload(k):
            ci, b = divmod(k, B)
            row = b * S + base + ci * CH
            return pltpu.async_copy(
                x_hbm.at[pl.ds(row, CH)], xb.at[k % 2], sxs[k % 2])

        hx, he, ho = {}, {}, {}
        he[0] = embload(0)
        hx[0] = xload(0)
        for k in range(nsteps):
            ci, b = divmod(k, B)
            if k + 1 < nsteps:
                if k >= 1:
                    ho[k - 1].wait()
                hx[k + 1] = xload(k + 1)
            if b == 0 and ci + 1 < nch:
                he[ci + 1] = embload(ci + 1)
            hx[k].wait()
            if b == 0:
                he[ci].wait()
            xbk = xb.at[k % 2]
            ebk = eb.at[ci % 2]

            def addbody(j, carry):
                o = j * 16
                for r in range(CH):
                    xbk[r, pl.ds(o, 16)] = (
                        xbk[r, pl.ds(o, 16)] + ebk[r, pl.ds(o, 16)])
                return carry

            lax.fori_loop(0, D // 16, addbody, 0)
            row = b * S + base + ci * CH
            ho[k] = pltpu.async_copy(
                xbk, out_hbm.at[pl.ds(row, CH)], sos[k % 2])
        ho[nsteps - 2].wait()
        ho[nsteps - 1].wait()

    return sc_add(x2, emb_table).reshape(B, S, D)


# speedup vs baseline: 2.2468x; 1.0001x over previous
import functools

import jax
import jax.numpy as jnp
from jax import lax
from jax.experimental import pallas as pl
from jax.experimental.pallas import tpu as pltpu
from jax.experimental.pallas import tpu_sc as plsc


_BS = 2048


def _tc_body(x_ref, emb_ref, out_ref):
    out_ref[...] = x_ref[...] + emb_ref[...][None, :, :]


def _tc_add(x, emb_table, nb):
    B, S, D = x.shape
    return pl.pallas_call(
        _tc_body,
        grid=(S // _BS, nb),
        in_specs=[
            pl.BlockSpec((1, _BS, D), lambda i, j: (j, i, 0)),
            pl.BlockSpec((_BS, D), lambda i, j: (i, 0)),
        ],
        out_specs=pl.BlockSpec((1, _BS, D), lambda i, j: (j, i, 0)),
        out_shape=jax.ShapeDtypeStruct((nb, S, D), x.dtype),
    )(x, emb_table)


def _sc_add(x2, emb_table, row0):
    S, D = emb_table.shape
    info = plsc.get_sparse_core_info()
    NC, NS = info.num_cores, info.num_subcores
    NW = NC * NS
    pos_per_w = S // NW
    CH = 16
    nch = pos_per_w // CH
    mesh = plsc.VectorSubcoreMesh(core_axis_name="c", subcore_axis_name="s")

    @functools.partial(
        pl.kernel,
        mesh=mesh,
        out_type=jax.ShapeDtypeStruct((S, D), jnp.float32),
        compiler_params=pltpu.CompilerParams(use_tc_tiling_on_sc=True),
        scratch_types=[
            pltpu.VMEM((2, CH, D), jnp.float32),
            pltpu.VMEM((2, CH, D), jnp.float32),
            pltpu.SemaphoreType.DMA,
            pltpu.SemaphoreType.DMA,
            pltpu.SemaphoreType.DMA,
            pltpu.SemaphoreType.DMA,
            pltpu.SemaphoreType.DMA,
            pltpu.SemaphoreType.DMA,
        ],
    )
    def body(x_hbm, emb_hbm, out_hbm, eb, xb, se0, se1, sx0, sx1, so0, so1):
        cid = lax.axis_index("c")
        sid = lax.axis_index("s")
        wid = sid * NC + cid
        base = wid * pos_per_w
        sxs, sos, ses = (sx0, sx1), (so0, so1), (se0, se1)

        def embload(ci):
            return pltpu.async_copy(
                emb_hbm.at[pl.ds(base + ci * CH, CH)], eb.at[ci % 2],
                ses[ci % 2])

        def xload(k):
            return pltpu.async_copy(
                x_hbm.at[pl.ds(row0 + base + k * CH, CH)], xb.at[k % 2],
                sxs[k % 2])

        hx, he, ho = {}, {}, {}
        he[0] = embload(0)
        hx[0] = xload(0)
        for k in range(nch):
            if k + 1 < nch:
                if k >= 1:
                    ho[k - 1].wait()
                hx[k + 1] = xload(k + 1)
                he[k + 1] = embload(k + 1)
            hx[k].wait()
            he[k].wait()
            xbk = xb.at[k % 2]
            ebk = eb.at[k % 2]

            def addbody(j, carry):
                o = j * 16
                for r in range(CH):
                    xbk[r, pl.ds(o, 16)] = (
                        xbk[r, pl.ds(o, 16)] + ebk[r, pl.ds(o, 16)])
                return carry

            lax.fori_loop(0, D // 16, addbody, 0)
            ho[k] = pltpu.async_copy(
                xbk, out_hbm.at[pl.ds(base + k * CH, CH)], sos[k % 2])
        ho[nch - 2].wait()
        ho[nch - 1].wait()

    return body(x2, emb_table)


def kernel(x, emb_table):
    B, S, D = x.shape
    x2 = x.reshape(B * S, D)
    sc_out = _sc_add(x2, emb_table, (B - 1) * S)
    tc_out = _tc_add(x, emb_table, B - 1)
    return jnp.concatenate([tc_out, sc_out[None]], axis=0)


# speedup vs baseline: 4.9789x; 2.2160x over previous
import jax
import jax.numpy as jnp
from jax.experimental import pallas as pl
from jax.experimental.pallas import tpu as pltpu


_BS = 2048


def _add_kernel(x_ref, emb_ref, out_ref):
    out_ref[...] = x_ref[...] + emb_ref[...][None, :, :]


def kernel(x, emb_table):
    B, S, D = x.shape
    num_s = S // _BS
    return pl.pallas_call(
        _add_kernel,
        grid=(num_s, B),
        compiler_params=pltpu.CompilerParams(
            dimension_semantics=("parallel", "arbitrary")),
        in_specs=[
            pl.BlockSpec((1, _BS, D), lambda i, j: (j, i, 0)),
            pl.BlockSpec((_BS, D), lambda i, j: (i, 0)),
        ],
        out_specs=pl.BlockSpec((1, _BS, D), lambda i, j: (j, i, 0)),
        out_shape=jax.ShapeDtypeStruct((B, S, D), x.dtype),
    )(x, emb_table)


# speedup vs baseline: 4.9876x; 1.0018x over previous
import jax
import jax.numpy as jnp
from jax.experimental import pallas as pl


_BS = 2048


def _add_kernel(x_ref, emb_ref, out_ref):
    out_ref[...] = x_ref[...] + emb_ref[...][None, :, :]


def kernel(x, emb_table):
    B, S, D = x.shape
    num_s = S // _BS
    return pl.pallas_call(
        _add_kernel,
        grid=(num_s, B),
        in_specs=[
            pl.BlockSpec((1, _BS, D), lambda i, j: (j, i, 0)),
            pl.BlockSpec((_BS, D), lambda i, j: (i, 0)),
        ],
        out_specs=pl.BlockSpec((1, _BS, D), lambda i, j: (j, i, 0)),
        out_shape=jax.ShapeDtypeStruct((B, S, D), x.dtype),
    )(x, emb_table)
